# baseline (speedup 1.0000x reference)
import functools

import jax
import jax.numpy as jnp
from jax import lax
from jax.experimental import pallas as pl
from jax.experimental.pallas import tpu as pltpu
from jax.experimental.pallas import tpu_sc as plsc

N = 16384
D = 128
K = 819
NBLK = 32
RPB = 512
KPAD = 1312

_INT_MIN = -2147483648


def _cvec(val):
    return jnp.full((16,), val, jnp.int32)


def _to_key(f):
    b = lax.bitcast_convert_type(f, jnp.int32)
    return jnp.where(b < 0, ~b, b ^ _INT_MIN)


def _srl(x, amount):
    return lax.shift_right_logical(x, _cvec(amount))


def _scores_body(x_ref, wgm_ref, bg_ref, s_ref):
    xb = x_ref[...]
    s = jax.lax.dot_general(xb, wgm_ref[...], (((1,), (0,)), ((), ())))
    s_ref[...] = (s[:, 0] + bg_ref[0, 0]).reshape(8, 128)


def _scores_tc(x, Wg, bg):
    wgm = jnp.zeros((D, D), jnp.float32).at[:, 0].set(Wg[0])
    out = pl.pallas_call(
        _scores_body,
        grid=(16,),
        in_specs=[
            pl.BlockSpec((1024, 128), lambda i: (i, 0)),
            pl.BlockSpec((128, 128), lambda i: (0, 0)),
            pl.BlockSpec(memory_space=pltpu.SMEM),
        ],
        out_specs=pl.BlockSpec((8, 128), lambda i: (i, 0)),
        out_shape=jax.ShapeDtypeStruct((128, 128), jnp.float32),
    )(x, wgm, bg.reshape(1, 1))
    return out.reshape(N)


def _select_body(scores_hbm, x_hbm,
                 mask_hbm, ax_hbm, cnt_hbm,
                 sc_v, key_v, hist_v, histall_v, rowids_v, idxc_v, rows_v,
                 maskb_v, cnt_v, cntall_v, cntrow_v,
                 sh_hist, sh_cnt, sem):
    c = lax.axis_index("c")
    s = lax.axis_index("s")
    j0 = 2 * s + c
    iota = lax.iota(jnp.int32, 16)
    zeros16 = jnp.zeros((16,), jnp.int32)

    pltpu.sync_copy(
        scores_hbm.at[pl.ds(pl.multiple_of(s * 1024, 1024), 1024)], sc_v)

    def key_body(v, _):
        key_v[pl.ds(v * 16, 16)] = _to_key(sc_v[pl.ds(v * 16, 16)])
        return 0
    lax.fori_loop(0, 64, key_body, 0)

    rk = jnp.int32(K)
    pref = jnp.int32(0)
    for p in (3, 2, 1, 0):
        def zero_body(g, _):
            hist_v[pl.ds(g * 16, 16)] = zeros16
            return 0
        lax.fori_loop(0, 16, zero_body, 0)

        def hist_body(v, _, _p=p, _pref=pref):
            kb = key_v[pl.ds(v * 16, 16)]
            digit = _srl(kb, 8 * _p) & _cvec(255)
            if _p == 3:
                act = jnp.full((16,), True)
            else:
                act = _srl(kb, 8 * _p + 8) == _pref
            cnt, last = plsc.scan_count(digit, mask=act)
            plsc.addupdate_scatter(hist_v, [digit], cnt, mask=last)
            return 0
        lax.fori_loop(0, 64, hist_body, 0)

        pltpu.sync_copy(hist_v, sh_hist.at[p * 16 + s])
        plsc.subcore_barrier()
        pltpu.sync_copy(sh_hist.at[pl.ds(p * 16, 16)], histall_v)

        tots = []
        for g in range(16):
            def merge_body(t, tot, _g=g):
                return tot + histall_v[t, pl.ds(_g * 16, 16)]
            tots.append(lax.fori_loop(0, 16, merge_body, zeros16))
        sufs = [None] * 16
        carry = jnp.int32(0)
        for g in range(15, -1, -1):
            r = lax.rev(tots[g], (0,))
            sufs[g] = lax.rev(plsc.cumsum(r), (0,)) + carry
            carry = carry + jnp.sum(tots[g])
        dstar = jnp.int32(-1)
        for g in range(16):
            idxg = iota + g * 16
            dstar = jnp.maximum(
                dstar, jnp.max(jnp.where(sufs[g] >= rk, idxg, -1)))
        cntgt = jnp.int32(0)
        for g in range(16):
            idxg = iota + g * 16
            cntgt = cntgt + jnp.sum(jnp.where(idxg > dstar, tots[g], 0))
        rk = rk - cntgt
        pref = lax.shift_left(pref, jnp.int32(8)) | dstar

    T = pref
    Tm = T ^ _INT_MIN

    half_counts = []
    for h in (0, 1):
        def cnt_body(v, accs, _h=h):
            gta, tia = accs
            kb = key_v[pl.ds(_h * 512 + v * 16, 16)]
            m = kb ^ _INT_MIN
            gta = gta + jnp.where(m > Tm, 1, 0)
            tia = tia + jnp.where(kb == T, 1, 0)
            return gta, tia
        gta, tia = lax.fori_loop(0, 32, cnt_body, (zeros16, zeros16))
        half_counts.append((jnp.sum(gta), jnp.sum(tia)))
    cv = (jnp.where(iota == 0, half_counts[0][0], 0)
          + jnp.where(iota == 1, half_counts[0][1], 0)
          + jnp.where(iota == 2, half_counts[1][0], 0)
          + jnp.where(iota == 3, half_counts[1][1], 0))
    def cpad_body(q, _):
        cnt_v[pl.ds(q * 16, 16)] = jnp.zeros((16,), jnp.int32)
        return 0
    lax.fori_loop(1, 16, cpad_body, 0)
    cnt_v[pl.ds(0, 16)] = cv
    pltpu.sync_copy(cnt_v, sh_cnt.at[s])
    plsc.subcore_barrier()
    pltpu.sync_copy(sh_cnt, cntall_v)

    tie_acc = jnp.int32(0)
    off_acc = jnp.int32(0)
    my_quota = jnp.int32(0)
    my_n = jnp.int32(0)
    my_off = jnp.int32(0)
    for sp in range(16):
        row = cntall_v[sp, pl.ds(0, 16)]
        for h in range(2):
            j = 2 * sp + h
            gt_j = jnp.sum(jnp.where(iota == 2 * h, row, 0))
            tie_j = jnp.sum(jnp.where(iota == 2 * h + 1, row, 0))
            quota_j = jnp.clip(rk - tie_acc, 0, tie_j)
            n_j = gt_j + quota_j
            pad_j = ((n_j + 15) // 16) * 16
            is_me = j0 == j
            my_quota = jnp.where(is_me, quota_j, my_quota)
            my_n = jnp.where(is_me, n_j, my_n)
            my_off = jnp.where(is_me, off_acc, my_off)
            tie_acc = tie_acc + tie_j
            off_acc = off_acc + pad_j

    base = j0 * 512
    basev = jnp.zeros((16,), jnp.int32) + base

    def fill_body(q, _):
        rowids_v[pl.ds(q * 16, 16)] = basev
        return 0
    lax.fori_loop(0, 33, fill_body, 0)

    def sel_body(v, carrys):
        tiec, selc = carrys
        kb = key_v[pl.ds(c * 512 + v * 16, 16)]
        m = kb ^ _INT_MIN
        tie = kb == T
        gt = m > Tm
        ind = jnp.where(tie, 1, 0)
        rank = tiec + plsc.cumsum(ind) - 1
        sel = gt | (tie & (rank < my_quota))
        maskb_v[pl.ds(v * 16, 16)] = jnp.where(sel, 1.0, 0.0)
        plsc.store_compressed(rowids_v.at[pl.ds(selc, 16)],
                              base + v * 16 + iota, mask=sel)
        return tiec + jnp.sum(ind), selc + jnp.sum(jnp.where(sel, 1, 0))
    lax.fori_loop(0, 32, sel_body, (jnp.int32(0), jnp.int32(0)))

    pltpu.sync_copy(maskb_v,
                    mask_hbm.at[pl.ds(pl.multiple_of(base, 512), 512)])
    cntrow_v[...] = jnp.where(iota == 0, my_n, 0)
    pltpu.sync_copy(cntrow_v,
                    cnt_hbm.at[pl.ds(pl.multiple_of(j0 * 16, 16), 16)])

    def gather_body(ci, _):
        idxc_v[...] = rowids_v[pl.ds(ci * 16, 16)]
        pltpu.async_copy(x_hbm.at[idxc_v], rows_v, sem).wait()
        pltpu.sync_copy(
            rows_v,
            ax_hbm.at[pl.ds(pl.multiple_of(my_off + ci * 16, 16), 16)])
        return 0
    lax.fori_loop(0, (my_n + 15) // 16, gather_body, 0)


def _select_sc(scores, x):
    mesh = plsc.VectorSubcoreMesh(core_axis_name="c", subcore_axis_name="s")
    call = pl.kernel(
        _select_body,
        out_type=[
            jax.ShapeDtypeStruct((N,), jnp.float32),
            jax.ShapeDtypeStruct((KPAD, D), jnp.float32),
            jax.ShapeDtypeStruct((NBLK * 16,), jnp.int32),
        ],
        mesh=mesh,
        compiler_params=pltpu.CompilerParams(needs_layout_passes=False),
        scratch_types=[
            pltpu.VMEM((1024,), jnp.float32),
            pltpu.VMEM((1024,), jnp.int32),
            pltpu.VMEM((256,), jnp.int32),
            pltpu.VMEM((16, 256), jnp.int32),
            pltpu.VMEM((528,), jnp.int32),
            pltpu.VMEM((16,), jnp.int32),
            pltpu.VMEM((16, 128), jnp.float32),
            pltpu.VMEM((512,), jnp.float32),
            pltpu.VMEM((256,), jnp.int32),
            pltpu.VMEM((16, 256), jnp.int32),
            pltpu.VMEM((16,), jnp.int32),
            pltpu.VMEM_SHARED((64, 256), jnp.int32),
            pltpu.VMEM_SHARED((16, 256), jnp.int32),
            pltpu.SemaphoreType.DMA,
        ],
    )
    return call(scores, x)


def _mlp_body(a_ref, w1_ref, b1_ref, w2_ref, b2_ref, o_ref):
    a = a_ref[...]
    dn = (((1,), (1,)), ((), ()))
    h = jax.lax.dot_general(a, w1_ref[...], dn,
                            preferred_element_type=jnp.float32)
    h = jnp.maximum(h + b1_ref[...], 0.0)
    o = jax.lax.dot_general(h, w2_ref[...], dn,
                            preferred_element_type=jnp.float32)
    o_ref[...] = o + b2_ref[...]


def _mlp_tc(ax, W1, b1, W2, b2):
    return pl.pallas_call(
        _mlp_body,
        out_shape=jax.ShapeDtypeStruct((KPAD, D), jnp.float32),
    )(ax, W1, b1.reshape(1, D), W2, b2.reshape(1, D))


def _scatter_body(ao_hbm, mask_hbm, cnt_hbm,
                  out_hbm,
                  buf_v, chunk_v, maskb_v, rowids_v, cntall_v, sem):
    c = lax.axis_index("c")
    s = lax.axis_index("s")
    j0 = 2 * s + c
    base = j0 * 512
    iota = lax.iota(jnp.int32, 16)
    zf32 = jnp.zeros((16,), jnp.float32)

    pltpu.sync_copy(cnt_hbm, cntall_v)

    off_acc = jnp.int32(0)
    my_n = jnp.int32(0)
    my_off = jnp.int32(0)
    for j in range(32):
        row = cntall_v[pl.ds(j * 16, 16)]
        n_j = jnp.sum(jnp.where(iota == 0, row, 0))
        pad_j = ((n_j + 15) // 16) * 16
        is_me = j0 == j
        my_n = jnp.where(is_me, n_j, my_n)
        my_off = jnp.where(is_me, off_acc, my_off)
        off_acc = off_acc + pad_j

    pltpu.sync_copy(mask_hbm.at[pl.ds(pl.multiple_of(base, 512), 512)],
                    maskb_v)

    def fill_body(q, _):
        rowids_v[pl.ds(q * 16, 16)] = jnp.zeros((16,), jnp.int32)
        return 0
    lax.fori_loop(0, 33, fill_body, 0)

    def comp_body(v, selc):
        mk = maskb_v[pl.ds(16 * v, 16)] > 0.5
        plsc.store_compressed(rowids_v.at[pl.ds(selc, 16)],
                              v * 16 + iota, mask=mk)
        return selc + jnp.sum(jnp.where(mk, 1, 0))
    lax.fori_loop(0, 32, comp_body, jnp.int32(0))

    def zero_body(r, _):
        for t in range(8):
            buf_v[r, pl.ds(16 * t, 16)] = zf32
        return 0
    lax.fori_loop(0, 512, zero_body, 0)

    def chunk_body(ci, _):
        pltpu.sync_copy(
            ao_hbm.at[pl.ds(pl.multiple_of(my_off + ci * 16, 16), 16)],
            chunk_v)
        rl = rowids_v[pl.ds(ci * 16, 16)]
        for jj in range(16):
            r = jnp.sum(jnp.where(iota == jj, rl, 0))

            @pl.when(ci * 16 + jj < my_n)
            def _():
                for t in range(8):
                    buf_v[r, pl.ds(16 * t, 16)] = chunk_v[jj, pl.ds(16 * t, 16)]
        return 0
    lax.fori_loop(0, (my_n + 15) // 16, chunk_body, 0)

    pltpu.sync_copy(buf_v,
                    out_hbm.at[pl.ds(pl.multiple_of(base, 512), 512)])


def _scatter_sc(active_out, mask, counts):
    mesh = plsc.VectorSubcoreMesh(core_axis_name="c", subcore_axis_name="s")
    call = pl.kernel(
        _scatter_body,
        out_type=jax.ShapeDtypeStruct((N, D), jnp.float32),
        mesh=mesh,
        compiler_params=pltpu.CompilerParams(needs_layout_passes=False),
        scratch_types=[
            pltpu.VMEM((512, 128), jnp.float32),
            pltpu.VMEM((16, 128), jnp.float32),
            pltpu.VMEM((512,), jnp.float32),
            pltpu.VMEM((528,), jnp.int32),
            pltpu.VMEM((NBLK * 16,), jnp.int32),
            pltpu.SemaphoreType.DMA,
        ],
    )
    return call(active_out, mask, counts)


def kernel(x, W1, b1, W2, b2, Wg, bg):
    scores = _scores_tc(x, Wg, bg)
    mask, active_x, counts = _select_sc(scores, x)
    active_out = _mlp_tc(active_x, W1, b1, W2, b2)
    out = _scatter_sc(active_out, mask, counts)
    return out, mask


# speedup vs baseline: 1.1217x; 1.1217x over previous
import functools

import jax
import jax.numpy as jnp
from jax import lax
from jax.experimental import pallas as pl
from jax.experimental.pallas import tpu as pltpu
from jax.experimental.pallas import tpu_sc as plsc

N = 16384
D = 128
K = 819

_INT_MIN = -2147483648


def _cvec(val):
    return jnp.full((16,), val, jnp.int32)


def _to_key(f):
    b = lax.bitcast_convert_type(f, jnp.int32)
    return jnp.where(b < 0, ~b, b ^ _INT_MIN)


def _srl(x, amount):
    return lax.shift_right_logical(x, _cvec(amount))


def _scores_body(x_ref, wgm_ref, bg_ref, s_ref):
    xb = x_ref[...]
    s = jax.lax.dot_general(xb, wgm_ref[...], (((1,), (0,)), ((), ())))
    s_ref[...] = (s[:, 0] + bg_ref[0, 0]).reshape(8, 128)


def _scores_tc(x, Wg, bg):
    wgm = jnp.zeros((D, D), jnp.float32).at[:, 0].set(Wg[0])
    out = pl.pallas_call(
        _scores_body,
        grid=(16,),
        in_specs=[
            pl.BlockSpec((1024, 128), lambda i: (i, 0)),
            pl.BlockSpec((128, 128), lambda i: (0, 0)),
            pl.BlockSpec(memory_space=pltpu.SMEM),
        ],
        out_specs=pl.BlockSpec((8, 128), lambda i: (i, 0)),
        out_shape=jax.ShapeDtypeStruct((128, 128), jnp.float32),
    )(x, wgm, bg.reshape(1, 1))
    return out.reshape(N)


def _select_body(scores_hbm,
                 mask_hbm,
                 sc_v, key_v, hist_v, histall_v, maskb_v, cnt_v, cntall_v,
                 sh_hist, sh_cnt):
    c = lax.axis_index("c")
    s = lax.axis_index("s")
    j0 = 2 * s + c
    iota = lax.iota(jnp.int32, 16)
    zeros16 = jnp.zeros((16,), jnp.int32)

    pltpu.sync_copy(
        scores_hbm.at[pl.ds(pl.multiple_of(s * 1024, 1024), 1024)], sc_v)

    def key_body(v, _):
        key_v[pl.ds(v * 16, 16)] = _to_key(sc_v[pl.ds(v * 16, 16)])
        return 0
    lax.fori_loop(0, 64, key_body, 0)

    rk = jnp.int32(K)
    pref = jnp.int32(0)
    for p in (3, 2, 1, 0):
        def zero_body(g, _):
            hist_v[pl.ds(g * 16, 16)] = zeros16
            return 0
        lax.fori_loop(0, 16, zero_body, 0)

        def hist_body(v, _, _p=p, _pref=pref):
            kb = key_v[pl.ds(v * 16, 16)]
            digit = _srl(kb, 8 * _p) & _cvec(255)
            if _p == 3:
                act = jnp.full((16,), True)
            else:
                act = _srl(kb, 8 * _p + 8) == _pref
            cnt, last = plsc.scan_count(digit, mask=act)
            plsc.addupdate_scatter(hist_v, [digit], cnt, mask=last)
            return 0
        lax.fori_loop(0, 64, hist_body, 0)

        pltpu.sync_copy(hist_v, sh_hist.at[p * 16 + s])
        plsc.subcore_barrier()
        pltpu.sync_copy(sh_hist.at[pl.ds(p * 16, 16)], histall_v)

        def merge_body(t, accs):
            return tuple(accs[g] + histall_v[t, pl.ds(g * 16, 16)]
                         for g in range(16))
        tots = lax.fori_loop(0, 16, merge_body, (zeros16,) * 16)
        sufs = [None] * 16
        carry = jnp.int32(0)
        for g in range(15, -1, -1):
            r = lax.rev(tots[g], (0,))
            sufs[g] = lax.rev(plsc.cumsum(r), (0,)) + carry
            carry = carry + jnp.sum(tots[g])
        dstar = jnp.int32(-1)
        for g in range(16):
            idxg = iota + g * 16
            dstar = jnp.maximum(
                dstar, jnp.max(jnp.where(sufs[g] >= rk, idxg, -1)))
        cntgt = jnp.int32(0)
        for g in range(16):
            idxg = iota + g * 16
            cntgt = cntgt + jnp.sum(jnp.where(idxg > dstar, tots[g], 0))
        rk = rk - cntgt
        pref = lax.shift_left(pref, jnp.int32(8)) | dstar

    T = pref

    ties = []
    for h in (0, 1):
        def cnt_body(v, tia, _h=h):
            kb = key_v[pl.ds(_h * 512 + v * 16, 16)]
            return tia + jnp.where(kb == T, 1, 0)
        tia = lax.fori_loop(0, 32, cnt_body, zeros16)
        ties.append(jnp.sum(tia))
    cv = (jnp.where(iota == 0, ties[0], 0)
          + jnp.where(iota == 1, ties[1], 0))
    def cpad_body(q, _):
        cnt_v[pl.ds(q * 16, 16)] = zeros16
        return 0
    lax.fori_loop(1, 16, cpad_body, 0)
    cnt_v[pl.ds(0, 16)] = cv
    pltpu.sync_copy(cnt_v, sh_cnt.at[s])
    plsc.subcore_barrier()
    pltpu.sync_copy(sh_cnt, cntall_v)

    tie_acc = jnp.int32(0)
    my_quota = jnp.int32(0)
    for sp in range(16):
        row = cntall_v[sp, pl.ds(0, 16)]
        for h in range(2):
            j = 2 * sp + h
            tie_j = jnp.sum(jnp.where(iota == h, row, 0))
            quota_j = jnp.clip(rk - tie_acc, 0, tie_j)
            my_quota = jnp.where(j0 == j, quota_j, my_quota)
            tie_acc = tie_acc + tie_j

    base = j0 * 512
    Tm = T ^ _INT_MIN

    def sel_body(v, tiec):
        kb = key_v[pl.ds(c * 512 + v * 16, 16)]
        m = kb ^ _INT_MIN
        tie = kb == T
        ind = jnp.where(tie, 1, 0)
        rank = tiec + plsc.cumsum(ind) - 1
        sel = (m > Tm) | (tie & (rank < my_quota))
        maskb_v[pl.ds(v * 16, 16)] = jnp.where(sel, 1.0, 0.0)
        return tiec + jnp.sum(ind)
    lax.fori_loop(0, 32, sel_body, jnp.int32(0))

    pltpu.sync_copy(maskb_v,
                    mask_hbm.at[pl.ds(pl.multiple_of(base, 512), 512)])


def _select_sc(scores):
    mesh = plsc.VectorSubcoreMesh(core_axis_name="c", subcore_axis_name="s")
    call = pl.kernel(
        _select_body,
        out_type=jax.ShapeDtypeStruct((N,), jnp.float32),
        mesh=mesh,
        compiler_params=pltpu.CompilerParams(needs_layout_passes=False),
        scratch_types=[
            pltpu.VMEM((1024,), jnp.float32),
            pltpu.VMEM((1024,), jnp.int32),
            pltpu.VMEM((256,), jnp.int32),
            pltpu.VMEM((16, 256), jnp.int32),
            pltpu.VMEM((512,), jnp.float32),
            pltpu.VMEM((256,), jnp.int32),
            pltpu.VMEM((16, 256), jnp.int32),
            pltpu.VMEM_SHARED((64, 256), jnp.int32),
            pltpu.VMEM_SHARED((16, 256), jnp.int32),
        ],
    )
    return call(scores)


def _mlp_body(x_ref, m_ref, w1_ref, b1_ref, w2_ref, b2_ref, o_ref):
    xb = x_ref[...]
    dn = (((1,), (1,)), ((), ()))
    h = jax.lax.dot_general(xb, w1_ref[...], dn)
    h = jnp.maximum(h + b1_ref[...], 0.0)
    o = jax.lax.dot_general(h, w2_ref[...], dn) + b2_ref[...]
    row = lax.broadcasted_iota(jnp.int32, (128, 128), 0)
    col = lax.broadcasted_iota(jnp.int32, (128, 128), 1)
    eye = jnp.where(row == col, 1.0, 0.0)
    mb = m_ref[...]
    for q in range(8):
        mq = jnp.broadcast_to(mb[q:q + 1, :], (128, 128))
        M = jax.lax.dot_general(eye, mq, (((1,), (1,)), ((), ())))
        o_ref[128 * q:128 * (q + 1), :] = o[128 * q:128 * (q + 1), :] * M


def _mlp_tc(x, mask2d, W1, b1, W2, b2):
    return pl.pallas_call(
        _mlp_body,
        grid=(16,),
        in_specs=[
            pl.BlockSpec((1024, 128), lambda i: (i, 0)),
            pl.BlockSpec((8, 128), lambda i: (i, 0)),
            pl.BlockSpec((128, 128), lambda i: (0, 0)),
            pl.BlockSpec((1, 128), lambda i: (0, 0)),
            pl.BlockSpec((128, 128), lambda i: (0, 0)),
            pl.BlockSpec((1, 128), lambda i: (0, 0)),
        ],
        out_specs=pl.BlockSpec((1024, 128), lambda i: (i, 0)),
        out_shape=jax.ShapeDtypeStruct((N, D), jnp.float32),
    )(x, mask2d, W1, b1.reshape(1, D), W2, b2.reshape(1, D))


def kernel(x, W1, b1, W2, b2, Wg, bg):
    scores = _scores_tc(x, Wg, bg)
    mask = _select_sc(scores)
    out = _mlp_tc(x, mask.reshape(128, 128), W1, b1, W2, b2)
    return out, mask
